# baseline (speedup 1.0000x reference)
import functools

import jax
import jax.numpy as jnp
from jax.experimental import pallas as pl
from jax.experimental.pallas import tpu as pltpu


_SQRT_2_OVER_PI = 0.7978845608028654


def _gate_from(slab, w1t_ref, w2_ref, inv_n):
    pooled = jnp.sum(slab, axis=-1, keepdims=True) * inv_n
    h = jnp.sum(w1t_ref[...] * pooled, axis=0, keepdims=True)
    h = 0.5 * h * (1.0 + jnp.tanh(_SQRT_2_OVER_PI * (h + 0.044715 * (h * h * h))))
    g = jnp.sum(w2_ref[...] * h, axis=1, keepdims=True)
    return 0.5 * (1.0 + jnp.tanh(0.5 * g))


def _se3d_body(xa_ref, xb_ref, w1t_ref, w2_ref, oa_ref, ob_ref, *, inv_n):
    xa = xa_ref[0]
    xb = xb_ref[0]
    oa_ref[0] = xa * _gate_from(xa, w1t_ref, w2_ref, inv_n)
    ob_ref[0] = xb * _gate_from(xb, w1t_ref, w2_ref, inv_n)


def kernel(x, w1, w2):
    B, C, D, H, W = x.shape
    N = D * H * W
    hidden = w1.shape[0]
    hb = B // 2

    x3 = x.reshape(B, C, N)
    w1t = jnp.transpose(w1)

    oa, ob = pl.pallas_call(
        functools.partial(_se3d_body, inv_n=1.0 / N),
        out_shape=[jax.ShapeDtypeStruct((hb, C, N), x.dtype),
                   jax.ShapeDtypeStruct((hb, C, N), x.dtype)],
        grid=(hb,),
        in_specs=[
            pl.BlockSpec((1, C, N), lambda b: (b, 0, 0)),
            pl.BlockSpec((1, C, N), lambda b: (b + 8, 0, 0)),
            pl.BlockSpec((C, hidden), lambda b: (0, 0)),
            pl.BlockSpec((C, hidden), lambda b: (0, 0)),
        ],
        out_specs=[pl.BlockSpec((1, C, N), lambda b: (b, 0, 0)),
                   pl.BlockSpec((1, C, N), lambda b: (b, 0, 0))],
        compiler_params=pltpu.CompilerParams(
            dimension_semantics=("parallel",),
            vmem_limit_bytes=48 << 20,
        ),
    )(x3, x3, w1t, w2)
    out3 = jnp.concatenate([oa, ob], axis=0)
    return out3.reshape(B, C, D, H, W)


# speedup vs baseline: 1.3199x; 1.3199x over previous
import functools

import jax
import jax.numpy as jnp
from jax.experimental import pallas as pl
from jax.experimental.pallas import tpu as pltpu


_SQRT_2_OVER_PI = 0.7978845608028654


def _se3d_body(x_ref, w1t_ref, w2_ref, o_ref, *, inv_n):
    xf = x_ref[0].astype(jnp.float32)
    pooled = jnp.sum(xf, axis=-1, keepdims=True) * inv_n
    h = jnp.sum(w1t_ref[...] * pooled, axis=0, keepdims=True)
    h = 0.5 * h * (1.0 + jnp.tanh(_SQRT_2_OVER_PI * (h + 0.044715 * (h * h * h))))
    g = jnp.sum(w2_ref[...] * h, axis=1, keepdims=True)
    gate = 0.5 * (1.0 + jnp.tanh(0.5 * g))
    o_ref[0] = (xf * gate).astype(jnp.bfloat16)


def kernel(x, w1, w2):
    B, C, D, H, W = x.shape
    N = D * H * W
    hidden = w1.shape[0]

    xbf = x.reshape(B, C, N).astype(jnp.bfloat16)
    w1t = jnp.transpose(w1)

    out_bf = pl.pallas_call(
        functools.partial(_se3d_body, inv_n=1.0 / N),
        out_shape=jax.ShapeDtypeStruct((B, C, N), jnp.bfloat16),
        grid=(B,),
        in_specs=[
            pl.BlockSpec((1, C, N), lambda b: (b, 0, 0)),
            pl.BlockSpec((C, hidden), lambda b: (0, 0)),
            pl.BlockSpec((C, hidden), lambda b: (0, 0)),
        ],
        out_specs=pl.BlockSpec((1, C, N), lambda b: (b, 0, 0)),
        compiler_params=pltpu.CompilerParams(
            dimension_semantics=("parallel",),
            vmem_limit_bytes=40 << 20,
        ),
    )(xbf, w1t, w2)
    return out_bf.astype(jnp.float32).reshape(B, C, D, H, W)
